# baseline (speedup 1.0000x reference)
import jax
import jax.numpy as jnp
from jax.experimental import pallas as pl
from jax.experimental.pallas import tpu as pltpu

_B, _N = 8, 100000
_T = 8192
_NC = (_N + _T - 1) // _T
_INV_CNT = 1.0 / float(_B * _N)
_EPS = 1e-5


def _mlp_bn_max_kernel(pos_ref, feat_ref, w1p_ref, w1f_ref, w2_ref, w3_ref,
                       g1_ref, be1_ref, g2_ref, be2_ref, g3_ref, be3_ref,
                       out_ref, stat1, stat2, stat3, mx3, mn3):
    p = pl.program_id(0)
    b = pl.program_id(1)
    c = pl.program_id(2)
    last_c = c == _NC - 1

    def dot(w, x):
        return jax.lax.dot_general(
            w, x, (((1,), (0,)), ((), ())),
            preferred_element_type=jnp.float32)

    def lane_mask():
        lane = jax.lax.broadcasted_iota(jnp.int32, (1, _T), 1)
        return (c * _T + lane) < _N

    def affine(stat_ref, g_ref, be_ref):
        m = stat_ref[:, 0:1] * _INV_CNT
        var = stat_ref[:, 1:2] * _INV_CNT - m * m
        a = g_ref[...] / jnp.sqrt(var + _EPS)
        return a, be_ref[...] - a * m

    def acc_stats(stat_ref, z):
        def upd(zm):
            s = jnp.sum(zm, axis=1, keepdims=True)
            q = jnp.sum(zm * zm, axis=1, keepdims=True)
            u = jnp.concatenate([s, q], axis=1)
            first = (b == 0) & (c == 0)
            stat_ref[...] = jnp.where(first, u, stat_ref[...] + u)

        @pl.when(jnp.logical_not(last_c))
        def _():
            upd(z)

        @pl.when(last_c)
        def _():
            upd(jnp.where(lane_mask(), z, 0.0))

    def z1_raw():
        return dot(w1p_ref[...], pos_ref[0]) + dot(w1f_ref[...], feat_ref[0])

    def h1():
        a1, c1 = affine(stat1, g1_ref, be1_ref)
        z = dot(a1 * w1p_ref[...], pos_ref[0]) + dot(a1 * w1f_ref[...], feat_ref[0])
        return jnp.maximum(z + c1, 0.0)

    @pl.when(p == 0)
    def _():
        acc_stats(stat1, z1_raw())

    @pl.when(p == 1)
    def _():
        acc_stats(stat2, dot(w2_ref[...], h1()))

    @pl.when(p == 2)
    def _():
        a2, c2 = affine(stat2, g2_ref, be2_ref)
        h2 = jnp.maximum(dot(a2 * w2_ref[...], h1()) + c2, 0.0)
        z3 = dot(w3_ref[...], h2)
        acc_stats(stat3, z3)

        def upd_minmax(zmax, zmin):
            colm = jax.lax.broadcasted_iota(jnp.int32, (1, _B), 1) == b
            init = (b == 0) & (c == 0)
            mxv = jnp.where(init, -jnp.inf, mx3[...])
            mnv = jnp.where(init, jnp.inf, mn3[...])
            mx3[...] = jnp.maximum(mxv, jnp.where(colm, zmax, -jnp.inf))
            mn3[...] = jnp.minimum(mnv, jnp.where(colm, zmin, jnp.inf))

        @pl.when(jnp.logical_not(last_c))
        def _():
            upd_minmax(jnp.max(z3, axis=1, keepdims=True),
                       jnp.min(z3, axis=1, keepdims=True))

        @pl.when(last_c)
        def _():
            m = lane_mask()
            upd_minmax(jnp.max(jnp.where(m, z3, -jnp.inf), axis=1, keepdims=True),
                       jnp.min(jnp.where(m, z3, jnp.inf), axis=1, keepdims=True))

        @pl.when((b == _B - 1) & last_c)
        def _():
            a3, c3 = affine(stat3, g3_ref, be3_ref)
            pick = jnp.where(a3 >= 0.0, mx3[...], mn3[...])
            out_ref[...] = jnp.maximum(a3 * pick + c3, 0.0)


def kernel(points_position, points_feature, W1, b1, g1, be1,
           W2, b2, g2, be2, W3, b3, g3, be3):
    B, _, N = points_position.shape
    D = points_feature.shape[1]
    del b1, b2, b3
    w1p, w1f = W1[:, :3], W1[:, 3:]
    col = lambda v: v[:, None]
    c1, c2, c3 = W1.shape[0], W2.shape[0], W3.shape[0]

    const = lambda p_, b_, c_: (0, 0)
    out = pl.pallas_call(
        _mlp_bn_max_kernel,
        grid=(3, _B, _NC),
        in_specs=[
            pl.BlockSpec((1, 3, _T), lambda p_, b_, c_: (b_, 0, c_)),
            pl.BlockSpec((1, D, _T), lambda p_, b_, c_: (b_, 0, c_)),
            pl.BlockSpec((c1, 3), const),
            pl.BlockSpec((c1, D), const),
            pl.BlockSpec((c2, c1), const),
            pl.BlockSpec((c3, c2), const),
            pl.BlockSpec((c1, 1), const),
            pl.BlockSpec((c1, 1), const),
            pl.BlockSpec((c2, 1), const),
            pl.BlockSpec((c2, 1), const),
            pl.BlockSpec((c3, 1), const),
            pl.BlockSpec((c3, 1), const),
        ],
        out_specs=pl.BlockSpec((c3, _B), const),
        out_shape=jax.ShapeDtypeStruct((c3, _B), jnp.float32),
        scratch_shapes=[
            pltpu.VMEM((c1, 2), jnp.float32),
            pltpu.VMEM((c2, 2), jnp.float32),
            pltpu.VMEM((c3, 2), jnp.float32),
            pltpu.VMEM((c3, _B), jnp.float32),
            pltpu.VMEM((c3, _B), jnp.float32),
        ],
        compiler_params=pltpu.CompilerParams(
            dimension_semantics=("arbitrary", "arbitrary", "arbitrary")),
    )(points_position, points_feature, w1p, w1f, W2, W3,
      col(g1), col(be1), col(g2), col(be2), col(g3), col(be3))

    feat_out = out.T[:, :, None]
    pos_out = jnp.zeros((B, 3, 1), dtype=points_position.dtype)
    return (pos_out, feat_out)


# speedup vs baseline: 1.0856x; 1.0856x over previous
import jax
import jax.numpy as jnp
from jax.experimental import pallas as pl
from jax.experimental.pallas import tpu as pltpu

_B, _N = 8, 100000
_T = 8192
_NC = (_N + _T - 1) // _T
_INV_CNT = 1.0 / float(_B * _N)
_EPS = 1e-5


def _mlp_bn_max_kernel(pos_ref, feat_ref, w1p_ref, w1f_ref, w2_ref, w3_ref,
                       g1_ref, be1_ref, g2_ref, be2_ref, g3_ref, be3_ref,
                       out_ref, stat1, stat2, stat3, mx3, mn3,
                       w1ps, w1fs, c1s, w2s, c2s):
    p = pl.program_id(0)
    b = pl.program_id(1)
    c = pl.program_id(2)
    last_c = c == _NC - 1
    seg_start = (b == 0) & (c == 0)

    def dot(w, x):
        return jax.lax.dot_general(
            w, x, (((1,), (0,)), ((), ())),
            preferred_element_type=jnp.float32)

    def lane_mask():
        lane = jax.lax.broadcasted_iota(jnp.int32, (1, _T), 1)
        return (c * _T + lane) < _N

    def affine(stat_ref, g_ref, be_ref):
        m = stat_ref[:, 0:1] * _INV_CNT
        var = stat_ref[:, 1:2] * _INV_CNT - m * m
        a = g_ref[...] / jnp.sqrt(var + _EPS)
        return a, be_ref[...] - a * m

    @pl.when((p == 1) & seg_start)
    def _():
        a1, c1 = affine(stat1, g1_ref, be1_ref)
        w1ps[...] = a1 * w1p_ref[...]
        w1fs[...] = a1 * w1f_ref[...]
        c1s[...] = c1

    @pl.when((p == 2) & seg_start)
    def _():
        a2, c2 = affine(stat2, g2_ref, be2_ref)
        w2s[...] = a2 * w2_ref[...]
        c2s[...] = c2

    def acc_stats(stat_ref, z):
        def upd(zm):
            s = jnp.sum(zm, axis=1, keepdims=True)
            q = jnp.sum(zm * zm, axis=1, keepdims=True)
            u = jnp.concatenate([s, q], axis=1)
            stat_ref[...] = jnp.where(seg_start, u, stat_ref[...] + u)

        @pl.when(jnp.logical_not(last_c))
        def _():
            upd(z)

        @pl.when(last_c)
        def _():
            upd(jnp.where(lane_mask(), z, 0.0))

    def h1():
        z = dot(w1ps[...], pos_ref[0]) + dot(w1fs[...], feat_ref[0])
        return jnp.maximum(z + c1s[...], 0.0)

    @pl.when(p == 0)
    def _():
        z1 = dot(w1p_ref[...], pos_ref[0]) + dot(w1f_ref[...], feat_ref[0])
        acc_stats(stat1, z1)

    @pl.when(p == 1)
    def _():
        acc_stats(stat2, dot(w2_ref[...], h1()))

    @pl.when(p == 2)
    def _():
        h2 = jnp.maximum(dot(w2s[...], h1()) + c2s[...], 0.0)
        z3 = dot(w3_ref[...], h2)
        acc_stats(stat3, z3)

        def upd_minmax(zmax, zmin):
            colm = jax.lax.broadcasted_iota(jnp.int32, (1, _B), 1) == b
            mxv = jnp.where(seg_start, -jnp.inf, mx3[...])
            mnv = jnp.where(seg_start, jnp.inf, mn3[...])
            mx3[...] = jnp.maximum(mxv, jnp.where(colm, zmax, -jnp.inf))
            mn3[...] = jnp.minimum(mnv, jnp.where(colm, zmin, jnp.inf))

        @pl.when(jnp.logical_not(last_c))
        def _():
            upd_minmax(jnp.max(z3, axis=1, keepdims=True),
                       jnp.min(z3, axis=1, keepdims=True))

        @pl.when(last_c)
        def _():
            m = lane_mask()
            upd_minmax(jnp.max(jnp.where(m, z3, -jnp.inf), axis=1, keepdims=True),
                       jnp.min(jnp.where(m, z3, jnp.inf), axis=1, keepdims=True))

        @pl.when((b == _B - 1) & last_c)
        def _():
            a3, c3 = affine(stat3, g3_ref, be3_ref)
            pick = jnp.where(a3 >= 0.0, mx3[...], mn3[...])
            out_ref[...] = jnp.maximum(a3 * pick + c3, 0.0)


def kernel(points_position, points_feature, W1, b1, g1, be1,
           W2, b2, g2, be2, W3, b3, g3, be3):
    B, _, N = points_position.shape
    D = points_feature.shape[1]
    del b1, b2, b3
    w1p, w1f = W1[:, :3], W1[:, 3:]
    col = lambda v: v[:, None]
    c1, c2, c3 = W1.shape[0], W2.shape[0], W3.shape[0]

    const = lambda p_, b_, c_: (0, 0)
    out = pl.pallas_call(
        _mlp_bn_max_kernel,
        grid=(3, _B, _NC),
        in_specs=[
            pl.BlockSpec((1, 3, _T), lambda p_, b_, c_: (b_, 0, c_)),
            pl.BlockSpec((1, D, _T), lambda p_, b_, c_: (b_, 0, c_)),
            pl.BlockSpec((c1, 3), const),
            pl.BlockSpec((c1, D), const),
            pl.BlockSpec((c2, c1), const),
            pl.BlockSpec((c3, c2), const),
            pl.BlockSpec((c1, 1), const),
            pl.BlockSpec((c1, 1), const),
            pl.BlockSpec((c2, 1), const),
            pl.BlockSpec((c2, 1), const),
            pl.BlockSpec((c3, 1), const),
            pl.BlockSpec((c3, 1), const),
        ],
        out_specs=pl.BlockSpec((c3, _B), const),
        out_shape=jax.ShapeDtypeStruct((c3, _B), jnp.float32),
        scratch_shapes=[
            pltpu.VMEM((c1, 2), jnp.float32),
            pltpu.VMEM((c2, 2), jnp.float32),
            pltpu.VMEM((c3, 2), jnp.float32),
            pltpu.VMEM((c3, _B), jnp.float32),
            pltpu.VMEM((c3, _B), jnp.float32),
            pltpu.VMEM((c1, 3), jnp.float32),
            pltpu.VMEM((c1, D), jnp.float32),
            pltpu.VMEM((c1, 1), jnp.float32),
            pltpu.VMEM((c2, c1), jnp.float32),
            pltpu.VMEM((c2, 1), jnp.float32),
        ],
        compiler_params=pltpu.CompilerParams(
            dimension_semantics=("arbitrary", "arbitrary", "arbitrary")),
    )(points_position, points_feature, w1p, w1f, W2, W3,
      col(g1), col(be1), col(g2), col(be2), col(g3), col(be3))

    feat_out = out.T[:, :, None]
    pos_out = jnp.zeros((B, 3, 1), dtype=points_position.dtype)
    return (pos_out, feat_out)


# speedup vs baseline: 1.1756x; 1.0828x over previous
import jax
import jax.numpy as jnp
from jax.experimental import pallas as pl
from jax.experimental.pallas import tpu as pltpu

_B, _N = 8, 100000
_T = 8192
_NC = (_N + _T - 1) // _T
_INV_CNT = 1.0 / float(_B * _N)
_EPS = 1e-5


def _mlp_bn_max_kernel(pos_ref, feat_ref, w1p_ref, w1f_ref, w2_ref, w3_ref,
                       g1_ref, be1_ref, g2_ref, be2_ref, g3_ref, be3_ref,
                       out_ref, stat1, stat2, stat3, mx3, mn3,
                       w1ps, w1fs, c1s, w2s, c2s):
    p = pl.program_id(0)
    b = pl.program_id(1)
    c = pl.program_id(2)
    last_c = c == _NC - 1
    seg_start = (b == 0) & (c == 0)

    def dot(w, x):
        return jax.lax.dot_general(
            w, x, (((1,), (0,)), ((), ())),
            preferred_element_type=jnp.float32)

    def lane_mask():
        lane = jax.lax.broadcasted_iota(jnp.int32, (1, _T), 1)
        return (c * _T + lane) < _N

    def affine(stat_ref, g_ref, be_ref):
        m = stat_ref[:, 0:1] * _INV_CNT
        var = stat_ref[:, 1:2] * _INV_CNT - m * m
        a = g_ref[...] / jnp.sqrt(var + _EPS)
        return a, be_ref[...] - a * m

    @pl.when((p == 1) & seg_start)
    def _():
        a1, c1 = affine(stat1, g1_ref, be1_ref)
        w1ps[...] = a1 * w1p_ref[...]
        w1fs[...] = a1 * w1f_ref[...]
        c1s[...] = c1

    @pl.when((p == 2) & seg_start)
    def _():
        a2, c2 = affine(stat2, g2_ref, be2_ref)
        w2s[...] = a2 * w2_ref[...]
        c2s[...] = c2

    def acc_stats(stat_ref, z):
        zm = jnp.where(lane_mask(), z, 0.0)
        s = jnp.sum(zm, axis=1, keepdims=True)
        q = jnp.sum(zm * zm, axis=1, keepdims=True)
        u = jnp.concatenate([s, q], axis=1)
        stat_ref[...] = jnp.where(seg_start, u, stat_ref[...] + u)

    def h1():
        z = dot(w1ps[...], pos_ref[0]) + dot(w1fs[...], feat_ref[0])
        return jnp.maximum(z + c1s[...], 0.0)

    @pl.when(p == 0)
    def _():
        z1 = dot(w1p_ref[...], pos_ref[0]) + dot(w1f_ref[...], feat_ref[0])
        acc_stats(stat1, z1)

    @pl.when(p == 1)
    def _():
        acc_stats(stat2, dot(w2_ref[...], h1()))

    @pl.when(p == 2)
    def _():
        h2 = jnp.maximum(dot(w2s[...], h1()) + c2s[...], 0.0)
        z3 = dot(w3_ref[...], h2)
        acc_stats(stat3, z3)

        m = lane_mask()
        zmax = jnp.max(jnp.where(m, z3, -jnp.inf), axis=1, keepdims=True)
        zmin = jnp.min(jnp.where(m, z3, jnp.inf), axis=1, keepdims=True)
        colm = jax.lax.broadcasted_iota(jnp.int32, (1, _B), 1) == b
        mxv = jnp.where(seg_start, -jnp.inf, mx3[...])
        mnv = jnp.where(seg_start, jnp.inf, mn3[...])
        mx3[...] = jnp.maximum(mxv, jnp.where(colm, zmax, -jnp.inf))
        mn3[...] = jnp.minimum(mnv, jnp.where(colm, zmin, jnp.inf))

        @pl.when((b == _B - 1) & last_c)
        def _():
            a3, c3 = affine(stat3, g3_ref, be3_ref)
            pick = jnp.where(a3 >= 0.0, mx3[...], mn3[...])
            out_ref[...] = jnp.maximum(a3 * pick + c3, 0.0)


def kernel(points_position, points_feature, W1, b1, g1, be1,
           W2, b2, g2, be2, W3, b3, g3, be3):
    B, _, N = points_position.shape
    D = points_feature.shape[1]
    del b1, b2, b3
    w1p, w1f = W1[:, :3], W1[:, 3:]
    col = lambda v: v[:, None]
    c1, c2, c3 = W1.shape[0], W2.shape[0], W3.shape[0]

    const = lambda p_, b_, c_: (0, 0)
    out = pl.pallas_call(
        _mlp_bn_max_kernel,
        grid=(3, _B, _NC),
        in_specs=[
            pl.BlockSpec((1, 3, _T), lambda p_, b_, c_: (b_, 0, c_)),
            pl.BlockSpec((1, D, _T), lambda p_, b_, c_: (b_, 0, c_)),
            pl.BlockSpec((c1, 3), const),
            pl.BlockSpec((c1, D), const),
            pl.BlockSpec((c2, c1), const),
            pl.BlockSpec((c3, c2), const),
            pl.BlockSpec((c1, 1), const),
            pl.BlockSpec((c1, 1), const),
            pl.BlockSpec((c2, 1), const),
            pl.BlockSpec((c2, 1), const),
            pl.BlockSpec((c3, 1), const),
            pl.BlockSpec((c3, 1), const),
        ],
        out_specs=pl.BlockSpec((c3, _B), const),
        out_shape=jax.ShapeDtypeStruct((c3, _B), jnp.float32),
        scratch_shapes=[
            pltpu.VMEM((c1, 2), jnp.float32),
            pltpu.VMEM((c2, 2), jnp.float32),
            pltpu.VMEM((c3, 2), jnp.float32),
            pltpu.VMEM((c3, _B), jnp.float32),
            pltpu.VMEM((c3, _B), jnp.float32),
            pltpu.VMEM((c1, 3), jnp.float32),
            pltpu.VMEM((c1, D), jnp.float32),
            pltpu.VMEM((c1, 1), jnp.float32),
            pltpu.VMEM((c2, c1), jnp.float32),
            pltpu.VMEM((c2, 1), jnp.float32),
        ],
        compiler_params=pltpu.CompilerParams(
            dimension_semantics=("arbitrary", "arbitrary", "arbitrary")),
    )(points_position, points_feature, w1p, w1f, W2, W3,
      col(g1), col(be1), col(g2), col(be2), col(g3), col(be3))

    feat_out = out.T[:, :, None]
    pos_out = jnp.zeros((B, 3, 1), dtype=points_position.dtype)
    return (pos_out, feat_out)


# speedup vs baseline: 1.3899x; 1.1823x over previous
import jax
import jax.numpy as jnp
from jax.experimental import pallas as pl
from jax.experimental.pallas import tpu as pltpu

_B, _N = 8, 100000
_T = 16384
_NC = (_N + _T - 1) // _T
_INV_CNT = 1.0 / float(_B * _N)
_EPS = 1e-5


def _mlp_bn_max_kernel(pos_ref, feat_ref, w1p_ref, w1f_ref, w2_ref, w3_ref,
                       g1_ref, be1_ref, g2_ref, be2_ref, g3_ref, be3_ref,
                       out_ref, stat1, stat2, stat3, mx3, mn3,
                       w1ps, w1fs, c1s, w2s, c2s):
    p = pl.program_id(0)
    b = pl.program_id(1)
    c = pl.program_id(2)
    last_c = c == _NC - 1
    seg_start = (b == 0) & (c == 0)

    def dot(w, x):
        return jax.lax.dot_general(
            w, x, (((1,), (0,)), ((), ())),
            preferred_element_type=jnp.float32)

    def lane_mask():
        lane = jax.lax.broadcasted_iota(jnp.int32, (1, _T), 1)
        return (c * _T + lane) < _N

    def affine(stat_ref, g_ref, be_ref):
        m = stat_ref[:, 0:1] * _INV_CNT
        var = stat_ref[:, 1:2] * _INV_CNT - m * m
        a = g_ref[...] / jnp.sqrt(var + _EPS)
        return a, be_ref[...] - a * m

    @pl.when((p == 1) & seg_start)
    def _():
        a1, c1 = affine(stat1, g1_ref, be1_ref)
        w1ps[...] = a1 * w1p_ref[...]
        w1fs[...] = a1 * w1f_ref[...]
        c1s[...] = c1

    @pl.when((p == 2) & seg_start)
    def _():
        a2, c2 = affine(stat2, g2_ref, be2_ref)
        w2s[...] = a2 * w2_ref[...]
        c2s[...] = c2

    def acc_stats(stat_ref, z):
        zm = jnp.where(lane_mask(), z, 0.0)
        s = jnp.sum(zm, axis=1, keepdims=True)
        q = jnp.sum(zm * zm, axis=1, keepdims=True)
        u = jnp.concatenate([s, q], axis=1)
        stat_ref[...] = jnp.where(seg_start, u, stat_ref[...] + u)

    def h1():
        z = dot(w1ps[...], pos_ref[0]) + dot(w1fs[...], feat_ref[0])
        return jnp.maximum(z + c1s[...], 0.0)

    @pl.when(p == 0)
    def _():
        z1 = dot(w1p_ref[...], pos_ref[0]) + dot(w1f_ref[...], feat_ref[0])
        acc_stats(stat1, z1)

    @pl.when(p == 1)
    def _():
        acc_stats(stat2, dot(w2_ref[...], h1()))

    @pl.when(p == 2)
    def _():
        h2 = jnp.maximum(dot(w2s[...], h1()) + c2s[...], 0.0)
        z3 = dot(w3_ref[...], h2)
        acc_stats(stat3, z3)

        m = lane_mask()
        zmax = jnp.max(jnp.where(m, z3, -jnp.inf), axis=1, keepdims=True)
        zmin = jnp.min(jnp.where(m, z3, jnp.inf), axis=1, keepdims=True)
        colm = jax.lax.broadcasted_iota(jnp.int32, (1, _B), 1) == b
        mxv = jnp.where(seg_start, -jnp.inf, mx3[...])
        mnv = jnp.where(seg_start, jnp.inf, mn3[...])
        mx3[...] = jnp.maximum(mxv, jnp.where(colm, zmax, -jnp.inf))
        mn3[...] = jnp.minimum(mnv, jnp.where(colm, zmin, jnp.inf))

        @pl.when((b == _B - 1) & last_c)
        def _():
            a3, c3 = affine(stat3, g3_ref, be3_ref)
            pick = jnp.where(a3 >= 0.0, mx3[...], mn3[...])
            out_ref[...] = jnp.maximum(a3 * pick + c3, 0.0)


def kernel(points_position, points_feature, W1, b1, g1, be1,
           W2, b2, g2, be2, W3, b3, g3, be3):
    B, _, N = points_position.shape
    D = points_feature.shape[1]
    del b1, b2, b3
    w1p, w1f = W1[:, :3], W1[:, 3:]
    col = lambda v: v[:, None]
    c1, c2, c3 = W1.shape[0], W2.shape[0], W3.shape[0]

    const = lambda p_, b_, c_: (0, 0)
    out = pl.pallas_call(
        _mlp_bn_max_kernel,
        grid=(3, _B, _NC),
        in_specs=[
            pl.BlockSpec((1, 3, _T), lambda p_, b_, c_: (b_, 0, c_)),
            pl.BlockSpec((1, D, _T), lambda p_, b_, c_: (b_, 0, c_)),
            pl.BlockSpec((c1, 3), const),
            pl.BlockSpec((c1, D), const),
            pl.BlockSpec((c2, c1), const),
            pl.BlockSpec((c3, c2), const),
            pl.BlockSpec((c1, 1), const),
            pl.BlockSpec((c1, 1), const),
            pl.BlockSpec((c2, 1), const),
            pl.BlockSpec((c2, 1), const),
            pl.BlockSpec((c3, 1), const),
            pl.BlockSpec((c3, 1), const),
        ],
        out_specs=pl.BlockSpec((c3, _B), const),
        out_shape=jax.ShapeDtypeStruct((c3, _B), jnp.float32),
        scratch_shapes=[
            pltpu.VMEM((c1, 2), jnp.float32),
            pltpu.VMEM((c2, 2), jnp.float32),
            pltpu.VMEM((c3, 2), jnp.float32),
            pltpu.VMEM((c3, _B), jnp.float32),
            pltpu.VMEM((c3, _B), jnp.float32),
            pltpu.VMEM((c1, 3), jnp.float32),
            pltpu.VMEM((c1, D), jnp.float32),
            pltpu.VMEM((c1, 1), jnp.float32),
            pltpu.VMEM((c2, c1), jnp.float32),
            pltpu.VMEM((c2, 1), jnp.float32),
        ],
        compiler_params=pltpu.CompilerParams(
            dimension_semantics=("arbitrary", "arbitrary", "arbitrary")),
    )(points_position, points_feature, w1p, w1f, W2, W3,
      col(g1), col(be1), col(g2), col(be2), col(g3), col(be3))

    feat_out = out.T[:, :, None]
    pos_out = jnp.zeros((B, 3, 1), dtype=points_position.dtype)
    return (pos_out, feat_out)
